# initial kernel scaffold (unmeasured)
import jax
import jax.numpy as jnp
from jax import lax
from jax.experimental import pallas as pl
from jax.experimental.pallas import tpu as pltpu

B, H, D, BS = 8, 8, 64, 16
NPAGE_G = 128
NPAGE_L = 64
NBLK = 64
NKEY = NPAGE_G * BS
SCALE = D ** -0.5


def kernel(Q, K, V, bt, lens):
    def body(q_ref, k_ref, v_ref, bt_ref, lens_ref, out_ref,
             k_all, v_all, sem_ks, sem_kr, sem_vs, sem_vr):
        my_x = lax.axis_index("x")
        my_y = lax.axis_index("y")
        my_z = lax.axis_index("z")
        peer = (my_x, 1 - my_y, my_z)

        barrier = pltpu.get_barrier_semaphore()
        pl.semaphore_signal(barrier, inc=1, device_id=peer,
                            device_id_type=pl.DeviceIdType.MESH)
        pl.semaphore_wait(barrier, 1)

        off = my_y * NPAGE_L
        k_all[pl.ds(off, NPAGE_L)] = k_ref[...].astype(jnp.bfloat16)
        v_all[pl.ds(off, NPAGE_L)] = v_ref[...].astype(jnp.bfloat16)

        rk = pltpu.make_async_remote_copy(
            src_ref=k_all.at[pl.ds(off, NPAGE_L)],
            dst_ref=k_all.at[pl.ds(off, NPAGE_L)],
            send_sem=sem_ks, recv_sem=sem_kr,
            device_id=peer, device_id_type=pl.DeviceIdType.MESH,
        )
        rv = pltpu.make_async_remote_copy(
            src_ref=v_all.at[pl.ds(off, NPAGE_L)],
            dst_ref=v_all.at[pl.ds(off, NPAGE_L)],
            send_sem=sem_vs, recv_sem=sem_vr,
            device_id=peer, device_id_type=pl.DeviceIdType.MESH,
        )
        rk.start()
        rv.start()

        btm = bt_ref[...]
        pages = lax.broadcasted_iota(jnp.int32, (B, NBLK, NPAGE_G), 2)
        blk = lax.broadcasted_iota(jnp.int32, (B, NBLK, NPAGE_G), 1)
        lens_v = lens_ref[...].reshape(B, 1, 1)
        hit = (btm[:, :, None] == pages) & (blk < lens_v)
        counts = jnp.sum(hit.astype(jnp.float32), axis=1)
        counts_k = jnp.broadcast_to(
            counts[:, :, None], (B, NPAGE_G, BS)
        ).reshape(B, NKEY)

        rk.wait()
        rv.wait()

        kk = k_all[...].reshape(NKEY, H, D)
        vv = v_all[...].reshape(NKEY, H, D)
        q = q_ref[...].reshape(B, H, D).astype(jnp.bfloat16)

        s = jnp.einsum("bhd,khd->bhk", q, kk,
                       preferred_element_type=jnp.float32) * SCALE
        valid = counts_k > 0.0
        s = jnp.where(valid[:, None, :], s, -1e30)
        m = jnp.max(s, axis=2, keepdims=True)
        w = counts_k[:, None, :] * jnp.exp(s - m)
        l = jnp.sum(w, axis=2, keepdims=True)
        p = (w / l).astype(jnp.bfloat16)
        o = jnp.einsum("bhk,khd->bhd", p, vv,
                       preferred_element_type=jnp.float32)
        out_ref[...] = o.reshape(B, 1, H, D)

    return pl.pallas_call(
        body,
        out_shape=jax.ShapeDtypeStruct((B, 1, H, D), jnp.float32),
        in_specs=[
            pl.BlockSpec(memory_space=pltpu.VMEM),
            pl.BlockSpec(memory_space=pltpu.VMEM),
            pl.BlockSpec(memory_space=pltpu.VMEM),
            pl.BlockSpec(memory_space=pltpu.VMEM),
            pl.BlockSpec(memory_space=pltpu.VMEM),
        ],
        out_specs=pl.BlockSpec(memory_space=pltpu.VMEM),
        scratch_shapes=[
            pltpu.VMEM((NPAGE_G, BS, H, D), jnp.bfloat16),
            pltpu.VMEM((NPAGE_G, BS, H, D), jnp.bfloat16),
            pltpu.SemaphoreType.DMA,
            pltpu.SemaphoreType.DMA,
            pltpu.SemaphoreType.DMA,
            pltpu.SemaphoreType.DMA,
        ],
        compiler_params=pltpu.CompilerParams(collective_id=0),
    )(Q, K, V, bt, lens)


# baseline (device time: 82371 ns/iter reference)
import jax
import jax.numpy as jnp
from jax import lax
from jax.experimental import pallas as pl
from jax.experimental.pallas import tpu as pltpu

B, H, D, BS = 8, 8, 64, 16
NPAGE_G = 128
NPAGE_L = 64
NBLK = 64
NKEY = NPAGE_G * BS
SCALE = D ** -0.5


def kernel(Q, K, V, bt, lens):
    def body(q_ref, k_ref, v_ref, bt_ref, lens_ref, out_ref,
             k_all, v_all, sem_ks, sem_kr, sem_vs, sem_vr):
        my_x = lax.axis_index("x")
        my_y = lax.axis_index("y")
        my_z = lax.axis_index("z")
        peer = (my_x, 1 - my_y, my_z)

        barrier = pltpu.get_barrier_semaphore()
        pl.semaphore_signal(barrier, inc=1, device_id=peer,
                            device_id_type=pl.DeviceIdType.MESH)
        pl.semaphore_wait(barrier, 1)

        def start_exchange(off):
            k_all[off:off + NPAGE_L] = k_ref[...].astype(jnp.bfloat16)
            v_all[off:off + NPAGE_L] = v_ref[...].astype(jnp.bfloat16)
            rk = pltpu.make_async_remote_copy(
                src_ref=k_all.at[pl.ds(off, NPAGE_L)],
                dst_ref=k_all.at[pl.ds(off, NPAGE_L)],
                send_sem=sem_ks, recv_sem=sem_kr,
                device_id=peer, device_id_type=pl.DeviceIdType.MESH,
            )
            rv = pltpu.make_async_remote_copy(
                src_ref=v_all.at[pl.ds(off, NPAGE_L)],
                dst_ref=v_all.at[pl.ds(off, NPAGE_L)],
                send_sem=sem_vs, recv_sem=sem_vr,
                device_id=peer, device_id_type=pl.DeviceIdType.MESH,
            )
            rk.start()
            rv.start()
            return rk, rv

        @pl.when(my_y == 0)
        def _():
            rk, rv = start_exchange(0)
            rk.wait()
            rv.wait()

        @pl.when(my_y == 1)
        def _():
            rk, rv = start_exchange(NPAGE_L)
            rk.wait()
            rv.wait()

        btm = bt_ref[...]
        pages = lax.broadcasted_iota(jnp.int32, (B, NBLK, NPAGE_G), 2)
        blk = lax.broadcasted_iota(jnp.int32, (B, NBLK, NPAGE_G), 1)
        lens_v = lens_ref[...].reshape(B, 1, 1)
        hit = (btm[:, :, None] == pages) & (blk < lens_v)
        counts = jnp.sum(hit.astype(jnp.float32), axis=1)
        counts_k = jnp.broadcast_to(
            counts[:, :, None], (B, NPAGE_G, BS)
        ).reshape(B, NKEY)

        kk = k_all[...].reshape(NKEY, H, D)
        vv = v_all[...].reshape(NKEY, H, D)
        q = q_ref[...].reshape(B, H, D).astype(jnp.bfloat16)

        s = jnp.einsum("bhd,khd->bhk", q, kk,
                       preferred_element_type=jnp.float32) * SCALE
        valid = counts_k > 0.0
        s = jnp.where(valid[:, None, :], s, -1e30)
        m = jnp.max(s, axis=2, keepdims=True)
        w = counts_k[:, None, :] * jnp.exp(s - m)
        l = jnp.sum(w, axis=2, keepdims=True)
        p = (w / l).astype(jnp.bfloat16)
        o = jnp.einsum("bhk,khd->bhd", p, vv,
                       preferred_element_type=jnp.float32)
        out_ref[...] = o.reshape(B, 1, H, D)

    return pl.pallas_call(
        body,
        out_shape=jax.ShapeDtypeStruct((B, 1, H, D), jnp.float32),
        in_specs=[
            pl.BlockSpec(memory_space=pltpu.VMEM),
            pl.BlockSpec(memory_space=pltpu.VMEM),
            pl.BlockSpec(memory_space=pltpu.VMEM),
            pl.BlockSpec(memory_space=pltpu.VMEM),
            pl.BlockSpec(memory_space=pltpu.VMEM),
        ],
        out_specs=pl.BlockSpec(memory_space=pltpu.VMEM),
        scratch_shapes=[
            pltpu.VMEM((NPAGE_G, BS, H, D), jnp.bfloat16),
            pltpu.VMEM((NPAGE_G, BS, H, D), jnp.bfloat16),
            pltpu.SemaphoreType.DMA,
            pltpu.SemaphoreType.DMA,
            pltpu.SemaphoreType.DMA,
            pltpu.SemaphoreType.DMA,
        ],
        compiler_params=pltpu.CompilerParams(
            collective_id=0, vmem_limit_bytes=100 * 1024 * 1024
        ),
    )(Q, K, V, bt, lens)


# device time: 24031 ns/iter; 3.4277x vs baseline; 3.4277x over previous
import jax
import jax.numpy as jnp
from jax import lax
from jax.experimental import pallas as pl
from jax.experimental.pallas import tpu as pltpu

B, H, D, BS = 8, 8, 64, 16
NPAGE_L = 64
NBLK = 64
NKEY_L = NPAGE_L * BS
SCALE = D ** -0.5


def kernel(Q, K, V, bt, lens):
    def body(q_ref, k_ref, v_ref, bt_ref, lens_ref, out_ref,
             o_buf, ml_buf, sem_os, sem_or, sem_mls, sem_mlr):
        my_x = lax.axis_index("x")
        my_y = lax.axis_index("y")
        my_z = lax.axis_index("z")
        peer = (my_x, 1 - my_y, my_z)

        barrier = pltpu.get_barrier_semaphore()
        pl.semaphore_signal(barrier, inc=1, device_id=peer,
                            device_id_type=pl.DeviceIdType.MESH)
        pl.semaphore_wait(barrier, 1)

        off = my_y * NPAGE_L
        btm = bt_ref[...] - off
        pages = lax.broadcasted_iota(jnp.int32, (B, NBLK, NPAGE_L), 2)
        blk = lax.broadcasted_iota(jnp.int32, (B, NBLK, NPAGE_L), 1)
        lens_v = lens_ref[...].reshape(B, 1, 1)
        hit = (btm[:, :, None] == pages) & (blk < lens_v)
        counts = jnp.sum(hit.astype(jnp.float32), axis=1)
        counts_k = jnp.broadcast_to(
            counts[:, :, None], (B, NPAGE_L, BS)
        ).reshape(B, NKEY_L)

        kk = k_ref[...].astype(jnp.bfloat16).reshape(NKEY_L, H, D)
        vv = v_ref[...].astype(jnp.bfloat16).reshape(NKEY_L, H, D)
        q = q_ref[...].reshape(B, H, D).astype(jnp.bfloat16)
        s = jnp.einsum("bhd,khd->bhk", q, kk,
                       preferred_element_type=jnp.float32) * SCALE
        valid = counts_k > 0.0
        s = jnp.where(valid[:, None, :], s, -1e30)
        m = jnp.max(s, axis=2)
        w = counts_k[:, None, :] * jnp.exp(s - m[:, :, None])
        l = jnp.sum(w, axis=2)
        o = jnp.einsum("bhk,khd->bhd", w.astype(jnp.bfloat16), vv,
                       preferred_element_type=jnp.float32)

        o_buf[0] = o
        ml_buf[0, 0] = m
        ml_buf[0, 1] = l
        ro = pltpu.make_async_remote_copy(
            src_ref=o_buf.at[0], dst_ref=o_buf.at[1],
            send_sem=sem_os, recv_sem=sem_or,
            device_id=peer, device_id_type=pl.DeviceIdType.MESH,
        )
        rml = pltpu.make_async_remote_copy(
            src_ref=ml_buf.at[0], dst_ref=ml_buf.at[1],
            send_sem=sem_mls, recv_sem=sem_mlr,
            device_id=peer, device_id_type=pl.DeviceIdType.MESH,
        )
        ro.start()
        rml.start()
        ro.wait()
        rml.wait()

        o_p = o_buf[1]
        m_p = ml_buf[1, 0]
        l_p = ml_buf[1, 1]
        mm = jnp.maximum(m, m_p)
        a_s = jnp.exp(m - mm)
        a_p = jnp.exp(m_p - mm)
        l_tot = l * a_s + l_p * a_p
        o_tot = o * a_s[:, :, None] + o_p * a_p[:, :, None]
        out_ref[...] = (o_tot / l_tot[:, :, None]).reshape(B, 1, H, D)

    return pl.pallas_call(
        body,
        out_shape=jax.ShapeDtypeStruct((B, 1, H, D), jnp.float32),
        in_specs=[pl.BlockSpec(memory_space=pltpu.VMEM)] * 5,
        out_specs=pl.BlockSpec(memory_space=pltpu.VMEM),
        scratch_shapes=[
            pltpu.VMEM((2, B, H, D), jnp.float32),
            pltpu.VMEM((2, 2, B, H), jnp.float32),
            pltpu.SemaphoreType.DMA,
            pltpu.SemaphoreType.DMA,
            pltpu.SemaphoreType.DMA,
            pltpu.SemaphoreType.DMA,
        ],
        compiler_params=pltpu.CompilerParams(
            collective_id=0, vmem_limit_bytes=100 * 1024 * 1024
        ),
    )(Q, K, V, bt, lens)


# device time: 13728 ns/iter; 6.0002x vs baseline; 1.7505x over previous
import jax
import jax.numpy as jnp
from jax import lax
from jax.experimental import pallas as pl
from jax.experimental.pallas import tpu as pltpu

B, H, D, BS = 8, 8, 64, 16
NPAGE_L = 64
NBLK = 64
NKEY_L = NPAGE_L * BS
SCALE = D ** -0.5


def kernel(Q, K, V, bt, lens):
    def body(q_ref, k_ref, v_ref, bt_ref, lens_ref, out_ref,
             o_buf, ml_buf, sem_os, sem_or, sem_mls, sem_mlr):
        my_x = lax.axis_index("x")
        my_y = lax.axis_index("y")
        my_z = lax.axis_index("z")
        peer = (my_x, 1 - my_y, my_z)

        barrier = pltpu.get_barrier_semaphore()
        pl.semaphore_signal(barrier, inc=1, device_id=peer,
                            device_id_type=pl.DeviceIdType.MESH)
        pl.semaphore_wait(barrier, 1)

        off = my_y * NPAGE_L
        btm = bt_ref[...] - off
        pages = lax.broadcasted_iota(jnp.int32, (B, NBLK, NPAGE_L), 2)
        blk = lax.broadcasted_iota(jnp.int32, (B, NBLK, NPAGE_L), 1)
        lens_v = lens_ref[...].reshape(B, 1, 1)
        hit = (btm[:, :, None] == pages) & (blk < lens_v)
        counts = jnp.sum(hit.astype(jnp.float32), axis=1)
        counts_k = jnp.broadcast_to(
            counts[:, :, None], (B, NPAGE_L, BS)
        ).reshape(B, NKEY_L)

        kk = k_ref[...].astype(jnp.bfloat16).reshape(NKEY_L, H, D)
        vv = v_ref[...].astype(jnp.bfloat16).reshape(NKEY_L, H, D)
        q = q_ref[...].reshape(B, H, D).astype(jnp.bfloat16)
        s = jnp.einsum("bhd,khd->bhk", q, kk,
                       preferred_element_type=jnp.float32) * SCALE
        valid = counts_k > 0.0
        s = jnp.where(valid[:, None, :], s, -1e30)
        m = jnp.max(s, axis=2)
        w = counts_k[:, None, :] * jnp.exp(s - m[:, :, None])
        l = jnp.sum(w, axis=2)

        ml_buf[0, 0] = m
        ml_buf[0, 1] = l
        rml = pltpu.make_async_remote_copy(
            src_ref=ml_buf.at[0], dst_ref=ml_buf.at[1],
            send_sem=sem_mls, recv_sem=sem_mlr,
            device_id=peer, device_id_type=pl.DeviceIdType.MESH,
        )
        rml.start()

        hh1 = lax.broadcasted_iota(jnp.int32, (H, H), 0)
        hh2 = lax.broadcasted_iota(jnp.int32, (H, H), 1)
        eye = (hh1 == hh2).astype(jnp.bfloat16)
        p4 = w.astype(jnp.bfloat16)[:, :, None, :] * eye[None, :, :, None]
        vvt = jnp.transpose(vv, (1, 0, 2))
        o = jnp.dot(
            p4.reshape(B * H, H * NKEY_L),
            vvt.reshape(H * NKEY_L, D),
            preferred_element_type=jnp.float32,
        ).reshape(B, H, D)

        o_buf[0] = o
        ro = pltpu.make_async_remote_copy(
            src_ref=o_buf.at[0], dst_ref=o_buf.at[1],
            send_sem=sem_os, recv_sem=sem_or,
            device_id=peer, device_id_type=pl.DeviceIdType.MESH,
        )
        ro.start()
        ro.wait()
        rml.wait()

        o_p = o_buf[1]
        m_p = ml_buf[1, 0]
        l_p = ml_buf[1, 1]
        mm = jnp.maximum(m, m_p)
        a_s = jnp.exp(m - mm)
        a_p = jnp.exp(m_p - mm)
        l_tot = l * a_s + l_p * a_p
        o_tot = o * a_s[:, :, None] + o_p * a_p[:, :, None]
        out_ref[...] = (o_tot / l_tot[:, :, None]).reshape(B, 1, H, D)

    return pl.pallas_call(
        body,
        out_shape=jax.ShapeDtypeStruct((B, 1, H, D), jnp.float32),
        in_specs=[pl.BlockSpec(memory_space=pltpu.VMEM)] * 5,
        out_specs=pl.BlockSpec(memory_space=pltpu.VMEM),
        scratch_shapes=[
            pltpu.VMEM((2, B, H, D), jnp.float32),
            pltpu.VMEM((2, 2, B, H), jnp.float32),
            pltpu.SemaphoreType.DMA,
            pltpu.SemaphoreType.DMA,
            pltpu.SemaphoreType.DMA,
            pltpu.SemaphoreType.DMA,
        ],
        compiler_params=pltpu.CompilerParams(
            collective_id=0, vmem_limit_bytes=100 * 1024 * 1024
        ),
    )(Q, K, V, bt, lens)
